# baseline (speedup 1.0000x reference)
import jax
import jax.numpy as jnp
from jax import lax
from jax.experimental import pallas as pl
from jax.experimental.pallas import tpu as pltpu

PROMPT_LENGTH = 64
EMBED_SIZE = 2048
CHUNK = 256
NBUF = 8


def _body(tokens_ref, prompt_hbm, x_hbm, out_hbm,
          table_vmem, gath_vmem, bufs_vmem, sem_table, sem_pre,
          sem_in, sem_out):
    batch = x_hbm.shape[0]
    seq_len = x_hbm.shape[1]
    chunks_per_batch = seq_len // CHUNK
    n_chunks = batch * chunks_per_batch

    def in_copy(i):
        b, c = divmod(i, chunks_per_batch)
        return pltpu.make_async_copy(
            x_hbm.at[b, pl.ds(c * CHUNK, CHUNK)],
            bufs_vmem.at[i % NBUF],
            sem_in.at[i % NBUF])

    def out_copy(i):
        b, c = divmod(i, chunks_per_batch)
        return pltpu.make_async_copy(
            bufs_vmem.at[i % NBUF],
            out_hbm.at[b, pl.ds(PROMPT_LENGTH + c * CHUNK, CHUNK)],
            sem_out.at[i % NBUF])

    tcopy = pltpu.make_async_copy(prompt_hbm, table_vmem, sem_table)
    tcopy.start()
    for i in range(min(NBUF, n_chunks)):
        in_copy(i).start()

    tcopy.wait()
    tok = tokens_ref[...]
    cols = lax.broadcasted_iota(jnp.int32, (PROMPT_LENGTH, PROMPT_LENGTH), 1)
    one_hot = (tok == cols).astype(jnp.float32)
    gath_vmem[...] = lax.dot(one_hot, table_vmem[...],
                             precision=lax.Precision.HIGHEST,
                             preferred_element_type=jnp.float32)
    pre = [
        pltpu.make_async_copy(
            gath_vmem,
            out_hbm.at[b, pl.ds(0, PROMPT_LENGTH)],
            sem_pre.at[b])
        for b in range(batch)
    ]
    for c in pre:
        c.start()

    for i in range(n_chunks):
        in_copy(i).wait()
        out_copy(i).start()
        if i + NBUF < n_chunks:
            out_copy(i).wait()
            in_copy(i + NBUF).start()
    for i in range(max(0, n_chunks - NBUF), n_chunks):
        out_copy(i).wait()
    for c in pre:
        c.wait()


def kernel(embedded_input, prompt_embedding, prompt_tokens):
    batch, seq_len, emb = embedded_input.shape
    tokens_2d = prompt_tokens.reshape(PROMPT_LENGTH, 1)

    return pl.pallas_call(
        _body,
        in_specs=[
            pl.BlockSpec(memory_space=pltpu.MemorySpace.VMEM),
            pl.BlockSpec(memory_space=pltpu.MemorySpace.HBM),
            pl.BlockSpec(memory_space=pltpu.MemorySpace.HBM),
        ],
        out_specs=pl.BlockSpec(memory_space=pltpu.MemorySpace.HBM),
        out_shape=jax.ShapeDtypeStruct(
            (batch, PROMPT_LENGTH + seq_len, emb), jnp.float32),
        scratch_shapes=[
            pltpu.VMEM((PROMPT_LENGTH, EMBED_SIZE), jnp.float32),
            pltpu.VMEM((PROMPT_LENGTH, EMBED_SIZE), jnp.float32),
            pltpu.VMEM((NBUF, CHUNK, EMBED_SIZE), jnp.float32),
            pltpu.SemaphoreType.DMA,
            pltpu.SemaphoreType.DMA((batch,)),
            pltpu.SemaphoreType.DMA((NBUF,)),
            pltpu.SemaphoreType.DMA((NBUF,)),
        ],
    )(tokens_2d, prompt_embedding, embedded_input)


# speedup vs baseline: 1.1219x; 1.1219x over previous
import jax
import jax.numpy as jnp
from jax import lax
from jax.experimental import pallas as pl
from jax.experimental.pallas import tpu as pltpu

PROMPT_LENGTH = 64
EMBED_SIZE = 2048
CHUNK = 512
NBUF = 8


def _body(tokens_ref, prompt_hbm, x_hbm, out_hbm,
          table_vmem, gath_vmem, bufs_vmem, sem_table, sem_pre,
          sem_in, sem_out):
    batch = x_hbm.shape[0]
    seq_len = x_hbm.shape[1]
    chunks_per_batch = seq_len // CHUNK
    n_chunks = batch * chunks_per_batch

    def in_copy(i):
        b, c = divmod(i, chunks_per_batch)
        return pltpu.make_async_copy(
            x_hbm.at[b, pl.ds(c * CHUNK, CHUNK)],
            bufs_vmem.at[i % NBUF],
            sem_in.at[i % NBUF])

    def out_copy(i):
        b, c = divmod(i, chunks_per_batch)
        return pltpu.make_async_copy(
            bufs_vmem.at[i % NBUF],
            out_hbm.at[b, pl.ds(PROMPT_LENGTH + c * CHUNK, CHUNK)],
            sem_out.at[i % NBUF])

    tcopy = pltpu.make_async_copy(prompt_hbm, table_vmem, sem_table)
    tcopy.start()
    for i in range(min(NBUF, n_chunks)):
        in_copy(i).start()

    tcopy.wait()
    tok = tokens_ref[...]
    cols = lax.broadcasted_iota(jnp.int32, (PROMPT_LENGTH, PROMPT_LENGTH), 1)
    one_hot = (tok == cols).astype(jnp.float32)
    gath_vmem[...] = lax.dot(one_hot, table_vmem[...],
                             precision=lax.Precision.HIGHEST,
                             preferred_element_type=jnp.float32)
    pre = [
        pltpu.make_async_copy(
            gath_vmem,
            out_hbm.at[b, pl.ds(0, PROMPT_LENGTH)],
            sem_pre.at[b])
        for b in range(batch)
    ]
    for c in pre:
        c.start()

    for i in range(n_chunks):
        in_copy(i).wait()
        out_copy(i).start()
        if i + NBUF < n_chunks:
            out_copy(i).wait()
            in_copy(i + NBUF).start()
    for i in range(max(0, n_chunks - NBUF), n_chunks):
        out_copy(i).wait()
    for c in pre:
        c.wait()


def kernel(embedded_input, prompt_embedding, prompt_tokens):
    batch, seq_len, emb = embedded_input.shape
    tokens_2d = prompt_tokens.reshape(PROMPT_LENGTH, 1)

    return pl.pallas_call(
        _body,
        in_specs=[
            pl.BlockSpec(memory_space=pltpu.MemorySpace.VMEM),
            pl.BlockSpec(memory_space=pltpu.MemorySpace.HBM),
            pl.BlockSpec(memory_space=pltpu.MemorySpace.HBM),
        ],
        out_specs=pl.BlockSpec(memory_space=pltpu.MemorySpace.HBM),
        out_shape=jax.ShapeDtypeStruct(
            (batch, PROMPT_LENGTH + seq_len, emb), jnp.float32),
        scratch_shapes=[
            pltpu.VMEM((PROMPT_LENGTH, EMBED_SIZE), jnp.float32),
            pltpu.VMEM((PROMPT_LENGTH, EMBED_SIZE), jnp.float32),
            pltpu.VMEM((NBUF, CHUNK, EMBED_SIZE), jnp.float32),
            pltpu.SemaphoreType.DMA,
            pltpu.SemaphoreType.DMA((batch,)),
            pltpu.SemaphoreType.DMA((NBUF,)),
            pltpu.SemaphoreType.DMA((NBUF,)),
        ],
    )(tokens_2d, prompt_embedding, embedded_input)


# speedup vs baseline: 1.1434x; 1.0192x over previous
import jax
import jax.numpy as jnp
from jax import lax
from jax.experimental import pallas as pl
from jax.experimental.pallas import tpu as pltpu

PROMPT_LENGTH = 64
EMBED_SIZE = 2048
CHUNK = 1024
NBUF = 4


def _body(tokens_ref, prompt_hbm, x_hbm, out_hbm,
          table_vmem, gath_vmem, bufs_vmem, sem_table, sem_pre,
          sem_in, sem_out):
    batch = x_hbm.shape[0]
    seq_len = x_hbm.shape[1]
    chunks_per_batch = seq_len // CHUNK
    n_chunks = batch * chunks_per_batch

    def in_copy(i):
        b, c = divmod(i, chunks_per_batch)
        return pltpu.make_async_copy(
            x_hbm.at[b, pl.ds(c * CHUNK, CHUNK)],
            bufs_vmem.at[i % NBUF],
            sem_in.at[i % NBUF])

    def out_copy(i):
        b, c = divmod(i, chunks_per_batch)
        return pltpu.make_async_copy(
            bufs_vmem.at[i % NBUF],
            out_hbm.at[b, pl.ds(PROMPT_LENGTH + c * CHUNK, CHUNK)],
            sem_out.at[i % NBUF])

    tcopy = pltpu.make_async_copy(prompt_hbm, table_vmem, sem_table)
    tcopy.start()
    for i in range(min(NBUF, n_chunks)):
        in_copy(i).start()

    tcopy.wait()
    tok = tokens_ref[...]
    cols = lax.broadcasted_iota(jnp.int32, (PROMPT_LENGTH, PROMPT_LENGTH), 1)
    one_hot = (tok == cols).astype(jnp.float32)
    gath_vmem[...] = lax.dot(one_hot, table_vmem[...],
                             precision=lax.Precision.HIGHEST,
                             preferred_element_type=jnp.float32)
    pre = [
        pltpu.make_async_copy(
            gath_vmem,
            out_hbm.at[b, pl.ds(0, PROMPT_LENGTH)],
            sem_pre.at[b])
        for b in range(batch)
    ]
    for c in pre:
        c.start()

    for i in range(n_chunks):
        in_copy(i).wait()
        out_copy(i).start()
        if i + NBUF < n_chunks:
            out_copy(i).wait()
            in_copy(i + NBUF).start()
    for i in range(max(0, n_chunks - NBUF), n_chunks):
        out_copy(i).wait()
    for c in pre:
        c.wait()


def kernel(embedded_input, prompt_embedding, prompt_tokens):
    batch, seq_len, emb = embedded_input.shape
    tokens_2d = prompt_tokens.reshape(PROMPT_LENGTH, 1)

    return pl.pallas_call(
        _body,
        in_specs=[
            pl.BlockSpec(memory_space=pltpu.MemorySpace.VMEM),
            pl.BlockSpec(memory_space=pltpu.MemorySpace.HBM),
            pl.BlockSpec(memory_space=pltpu.MemorySpace.HBM),
        ],
        out_specs=pl.BlockSpec(memory_space=pltpu.MemorySpace.HBM),
        out_shape=jax.ShapeDtypeStruct(
            (batch, PROMPT_LENGTH + seq_len, emb), jnp.float32),
        scratch_shapes=[
            pltpu.VMEM((PROMPT_LENGTH, EMBED_SIZE), jnp.float32),
            pltpu.VMEM((PROMPT_LENGTH, EMBED_SIZE), jnp.float32),
            pltpu.VMEM((NBUF, CHUNK, EMBED_SIZE), jnp.float32),
            pltpu.SemaphoreType.DMA,
            pltpu.SemaphoreType.DMA((batch,)),
            pltpu.SemaphoreType.DMA((NBUF,)),
            pltpu.SemaphoreType.DMA((NBUF,)),
        ],
    )(tokens_2d, prompt_embedding, embedded_input)


# speedup vs baseline: 1.1659x; 1.0197x over previous
import jax
import jax.numpy as jnp
from jax import lax
from jax.experimental import pallas as pl
from jax.experimental.pallas import tpu as pltpu

PROMPT_LENGTH = 64
EMBED_SIZE = 2048
CHUNK = 1024
NBUF = 6


def _body(tokens_ref, prompt_hbm, x_hbm, out_hbm,
          table_vmem, gath_vmem, bufs_vmem, sem_table, sem_pre,
          sem_in, sem_out):
    batch = x_hbm.shape[0]
    seq_len = x_hbm.shape[1]
    chunks_per_batch = seq_len // CHUNK
    n_chunks = batch * chunks_per_batch

    def in_copy(i):
        b, c = divmod(i, chunks_per_batch)
        return pltpu.make_async_copy(
            x_hbm.at[b, pl.ds(c * CHUNK, CHUNK)],
            bufs_vmem.at[i % NBUF],
            sem_in.at[i % NBUF])

    def out_copy(i):
        b, c = divmod(i, chunks_per_batch)
        return pltpu.make_async_copy(
            bufs_vmem.at[i % NBUF],
            out_hbm.at[b, pl.ds(PROMPT_LENGTH + c * CHUNK, CHUNK)],
            sem_out.at[i % NBUF])

    tcopy = pltpu.make_async_copy(prompt_hbm, table_vmem, sem_table)
    tcopy.start()
    for i in range(min(NBUF, n_chunks)):
        in_copy(i).start()

    tcopy.wait()
    tok = tokens_ref[...]
    cols = lax.broadcasted_iota(jnp.int32, (PROMPT_LENGTH, PROMPT_LENGTH), 1)
    one_hot = (tok == cols).astype(jnp.float32)
    gath_vmem[...] = lax.dot(one_hot, table_vmem[...],
                             precision=lax.Precision.HIGHEST,
                             preferred_element_type=jnp.float32)
    pre = [
        pltpu.make_async_copy(
            gath_vmem,
            out_hbm.at[b, pl.ds(0, PROMPT_LENGTH)],
            sem_pre.at[b])
        for b in range(batch)
    ]
    for c in pre:
        c.start()

    for i in range(n_chunks):
        in_copy(i).wait()
        out_copy(i).start()
        if i + NBUF < n_chunks:
            out_copy(i).wait()
            in_copy(i + NBUF).start()
    for i in range(max(0, n_chunks - NBUF), n_chunks):
        out_copy(i).wait()
    for c in pre:
        c.wait()


def kernel(embedded_input, prompt_embedding, prompt_tokens):
    batch, seq_len, emb = embedded_input.shape
    tokens_2d = prompt_tokens.reshape(PROMPT_LENGTH, 1)

    return pl.pallas_call(
        _body,
        in_specs=[
            pl.BlockSpec(memory_space=pltpu.MemorySpace.VMEM),
            pl.BlockSpec(memory_space=pltpu.MemorySpace.HBM),
            pl.BlockSpec(memory_space=pltpu.MemorySpace.HBM),
        ],
        out_specs=pl.BlockSpec(memory_space=pltpu.MemorySpace.HBM),
        out_shape=jax.ShapeDtypeStruct(
            (batch, PROMPT_LENGTH + seq_len, emb), jnp.float32),
        scratch_shapes=[
            pltpu.VMEM((PROMPT_LENGTH, EMBED_SIZE), jnp.float32),
            pltpu.VMEM((PROMPT_LENGTH, EMBED_SIZE), jnp.float32),
            pltpu.VMEM((NBUF, CHUNK, EMBED_SIZE), jnp.float32),
            pltpu.SemaphoreType.DMA,
            pltpu.SemaphoreType.DMA((batch,)),
            pltpu.SemaphoreType.DMA((NBUF,)),
            pltpu.SemaphoreType.DMA((NBUF,)),
        ],
    )(tokens_2d, prompt_embedding, embedded_input)
